# baseline (speedup 1.0000x reference)
import functools

import jax
import jax.numpy as jnp
from jax import lax
from jax.experimental import pallas as pl
from jax.experimental.pallas import tpu as pltpu
from jax.experimental.pallas import tpu_sc as plsc

_B, _C, _H, _W = 8, 3, 512, 512
_K = 512
_TAB = 65536
_NC, _NS, _L = 2, 16, 16
_NW = _NC * _NS
_QPI = _NW // _B
_RPW = _H // _QPI
_CR = 8
_NCHUNK = _RPW // _CR
_NSUP = _NCHUNK // 2
_GRP = _CR * _W // _L


def _cluster_body(in_hbm, idx_hbm, crg_hbm, cb_hbm, out_hbm, lab_hbm,
                  tab, crg, cb,
                  ins_a, ins_b, outs_a, outs_b,
                  tsem, si_a, si_b, so_a, so_b):
    cid = lax.axis_index("c")
    sid = lax.axis_index("s")
    wid = sid * _NC + cid
    img = wid // _QPI
    row0 = (wid % _QPI) * _RPW

    def in_slices(k):
        r0 = row0 + k * _CR
        return [in_hbm.at[img, c, pl.ds(r0, _CR)] for c in range(_C)]

    def out_slices(k):
        r0 = row0 + k * _CR
        return [out_hbm.at[img, 0, pl.ds(r0, _CR)],
                out_hbm.at[img, 1, pl.ds(r0, _CR)],
                out_hbm.at[img, 2, pl.ds(r0, _CR)],
                lab_hbm.at[img, 0, pl.ds(r0, _CR)]]

    def start_in(k, bufs, sem):
        for src, dst in zip(in_slices(k), bufs):
            pltpu.async_copy(src, dst, sem)

    def wait_in(k, bufs, sem):
        for src, dst in zip(in_slices(k), bufs):
            pltpu.make_async_copy(src, dst, sem).wait()

    def start_out(k, bufs, sem):
        for src, dst in zip(bufs, out_slices(k)):
            pltpu.async_copy(src, dst, sem)

    def wait_out(k, bufs, sem):
        for src, dst in zip(bufs, out_slices(k)):
            pltpu.make_async_copy(src, dst, sem).wait()

    def compute(ins, outs):
        rb, gb, bb = ins
        orb, ogb, obb, lb = outs

        @plsc.parallel_loop(0, _GRP, unroll=4)
        def _(i):
            r = i >> 5
            s = pl.ds((i & 31) * _L, _L)
            r5 = (rb[r, s] * 31.875).astype(jnp.int32)
            g6 = (gb[r, s] * 63.75).astype(jnp.int32)
            b5 = (bb[r, s] * 31.875).astype(jnp.int32)
            code = (r5 << 11) | (g6 << 5) | b5
            label = plsc.load_gather(tab, [code])
            w = plsc.load_gather(crg, [label])
            orb[r, s] = plsc.bitcast(w << 16, jnp.float32)
            ogb[r, s] = plsc.bitcast(w & jnp.int32(-65536), jnp.float32)
            obb[r, s] = plsc.load_gather(cb, [label])
            lb[r, s] = label

    pltpu.async_copy(idx_hbm, tab, tsem)
    pltpu.async_copy(crg_hbm, crg, tsem)
    pltpu.async_copy(cb_hbm, cb, tsem)
    start_in(0, ins_a, si_a)
    pltpu.make_async_copy(idx_hbm, tab, tsem).wait()
    pltpu.make_async_copy(crg_hbm, crg, tsem).wait()
    pltpu.make_async_copy(cb_hbm, cb, tsem).wait()

    def super_body(t, carry):
        k0 = 2 * t
        k1 = k0 + 1
        start_in(k1, ins_b, si_b)
        wait_in(k0, ins_a, si_a)

        @pl.when(t > 0)
        def _():
            wait_out(k0, outs_a, so_a)

        compute(ins_a, outs_a)
        start_out(k0, outs_a, so_a)

        @pl.when(t < _NSUP - 1)
        def _():
            start_in(k0 + 2, ins_a, si_a)

        wait_in(k1, ins_b, si_b)

        @pl.when(t > 0)
        def _():
            wait_out(k1, outs_b, so_b)

        compute(ins_b, outs_b)
        start_out(k1, outs_b, so_b)
        return carry

    lax.fori_loop(0, _NSUP, super_body, 0)
    wait_out(_NCHUNK - 2, outs_a, so_a)
    wait_out(_NCHUNK - 1, outs_b, so_b)


_cluster_sc = functools.partial(
    pl.kernel,
    out_type=(
        jax.ShapeDtypeStruct((_B, _C, _H, _W), jnp.float32),
        jax.ShapeDtypeStruct((_B, 1, _H, _W), jnp.int32),
    ),
    mesh=plsc.VectorSubcoreMesh(core_axis_name="c", subcore_axis_name="s"),
    compiler_params=pltpu.CompilerParams(needs_layout_passes=False, skip_device_barrier=True),
    scratch_types=[
        pltpu.VMEM((_TAB,), jnp.int32),
        pltpu.VMEM((_K,), jnp.int32),
        pltpu.VMEM((_K,), jnp.float32),
        [pltpu.VMEM((_CR, _W), jnp.float32)] * 3,
        [pltpu.VMEM((_CR, _W), jnp.float32)] * 3,
        [pltpu.VMEM((_CR, _W), jnp.float32)] * 3
        + [pltpu.VMEM((_CR, _W), jnp.int32)],
        [pltpu.VMEM((_CR, _W), jnp.float32)] * 3
        + [pltpu.VMEM((_CR, _W), jnp.int32)],
        pltpu.SemaphoreType.DMA,
        pltpu.SemaphoreType.DMA,
        pltpu.SemaphoreType.DMA,
        pltpu.SemaphoreType.DMA,
        pltpu.SemaphoreType.DMA,
    ],
)(_cluster_body)


def kernel(input, index, center):
    r16 = lax.bitcast_convert_type(
        center[:, 0].astype(jnp.bfloat16), jnp.uint16).astype(jnp.uint32)
    g16 = lax.bitcast_convert_type(
        center[:, 1].astype(jnp.bfloat16), jnp.uint16).astype(jnp.uint32)
    crg = lax.bitcast_convert_type(r16 | (g16 << 16), jnp.int32)
    cbf = center[:, 2]
    output, label = _cluster_sc(input, index, crg, cbf)
    return (output, label)


# speedup vs baseline: 1.0022x; 1.0022x over previous
import functools

import jax
import jax.numpy as jnp
from jax import lax
from jax.experimental import pallas as pl
from jax.experimental.pallas import tpu as pltpu
from jax.experimental.pallas import tpu_sc as plsc

_B, _C, _H, _W = 8, 3, 512, 512
_K = 512
_TAB = 65536
_NC, _NS, _L = 2, 16, 16
_NW = _NC * _NS
_QPI = _NW // _B
_RPW = _H // _QPI
_CR = 8
_NCHUNK = _RPW // _CR
_NSUP = _NCHUNK // 2
_GRP = _CR * _W // _L


def _cluster_body(in_hbm, idx_hbm, crg_hbm, cb_hbm, out_hbm, lab_hbm,
                  tab, crg, cb,
                  ins_a, ins_b, outs_a, outs_b,
                  tsem, si_a, si_b, so_a, so_b):
    cid = lax.axis_index("c")
    sid = lax.axis_index("s")
    wid = sid * _NC + cid
    img = wid // _QPI
    row0 = (wid % _QPI) * _RPW

    def in_slices(k):
        r0 = row0 + k * _CR
        return [in_hbm.at[img, c, pl.ds(r0, _CR)] for c in range(_C)]

    def out_slices(k):
        r0 = row0 + k * _CR
        return [out_hbm.at[img, 0, pl.ds(r0, _CR)],
                out_hbm.at[img, 1, pl.ds(r0, _CR)],
                out_hbm.at[img, 2, pl.ds(r0, _CR)],
                lab_hbm.at[img, 0, pl.ds(r0, _CR)]]

    def start_in(k, bufs, sem):
        for src, dst in zip(in_slices(k), bufs):
            pltpu.async_copy(src, dst, sem)

    def wait_in(k, bufs, sem):
        for src, dst in zip(in_slices(k), bufs):
            pltpu.make_async_copy(src, dst, sem).wait()

    def start_out(k, bufs, sem):
        for src, dst in zip(bufs, out_slices(k)):
            pltpu.async_copy(src, dst, sem)

    def wait_out(k, bufs, sem):
        for src, dst in zip(bufs, out_slices(k)):
            pltpu.make_async_copy(src, dst, sem).wait()

    def compute(ins, outs):
        rb, gb, bb = ins
        orb, ogb, obb, lb = outs

        @plsc.parallel_loop(0, _GRP, unroll=4)
        def _(i):
            r = i >> 5
            s = pl.ds((i & 31) * _L, _L)
            r5 = (rb[r, s] * 31.875).astype(jnp.int32)
            g6 = (gb[r, s] * 63.75).astype(jnp.int32)
            b5 = (bb[r, s] * 31.875).astype(jnp.int32)
            code = (r5 << 11) | (g6 << 5) | b5
            label = plsc.load_gather(tab, [code])
            w = plsc.load_gather(crg, [label])
            orb[r, s] = plsc.bitcast(w << 16, jnp.float32)
            ogb[r, s] = plsc.bitcast(w & jnp.int32(-65536), jnp.float32)
            obb[r, s] = plsc.load_gather(cb, [label])
            lb[r, s] = label

    pltpu.async_copy(idx_hbm, tab, tsem)
    pltpu.async_copy(crg_hbm, crg, tsem)
    pltpu.async_copy(cb_hbm, cb, tsem)
    start_in(0, ins_a, si_a)
    pltpu.make_async_copy(idx_hbm, tab, tsem).wait()
    pltpu.make_async_copy(crg_hbm, crg, tsem).wait()
    pltpu.make_async_copy(cb_hbm, cb, tsem).wait()

    def super_body(t, carry):
        k0 = 2 * t
        k1 = k0 + 1
        start_in(k1, ins_b, si_b)
        wait_in(k0, ins_a, si_a)

        @pl.when(t > 0)
        def _():
            wait_out(k0, outs_a, so_a)

        compute(ins_a, outs_a)
        start_out(k0, outs_a, so_a)

        @pl.when(t < _NSUP - 1)
        def _():
            start_in(k0 + 2, ins_a, si_a)

        wait_in(k1, ins_b, si_b)

        @pl.when(t > 0)
        def _():
            wait_out(k1, outs_b, so_b)

        compute(ins_b, outs_b)
        start_out(k1, outs_b, so_b)
        return carry

    lax.fori_loop(0, _NSUP, super_body, 0)
    wait_out(_NCHUNK - 2, outs_a, so_a)
    wait_out(_NCHUNK - 1, outs_b, so_b)


_cluster_sc = functools.partial(
    pl.kernel,
    out_type=(
        jax.ShapeDtypeStruct((_B, _C, _H, _W), jnp.float32),
        jax.ShapeDtypeStruct((_B, 1, _H, _W), jnp.int32),
    ),
    mesh=plsc.VectorSubcoreMesh(core_axis_name="c", subcore_axis_name="s"),
    compiler_params=pltpu.CompilerParams(needs_layout_passes=False),
    scratch_types=[
        pltpu.VMEM((_TAB,), jnp.int32),
        pltpu.VMEM((_K,), jnp.int32),
        pltpu.VMEM((_K,), jnp.float32),
        [pltpu.VMEM((_CR, _W), jnp.float32)] * 3,
        [pltpu.VMEM((_CR, _W), jnp.float32)] * 3,
        [pltpu.VMEM((_CR, _W), jnp.float32)] * 3
        + [pltpu.VMEM((_CR, _W), jnp.int32)],
        [pltpu.VMEM((_CR, _W), jnp.float32)] * 3
        + [pltpu.VMEM((_CR, _W), jnp.int32)],
        pltpu.SemaphoreType.DMA,
        pltpu.SemaphoreType.DMA,
        pltpu.SemaphoreType.DMA,
        pltpu.SemaphoreType.DMA,
        pltpu.SemaphoreType.DMA,
    ],
)(_cluster_body)


def kernel(input, index, center):
    r16 = lax.bitcast_convert_type(
        center[:, 0].astype(jnp.bfloat16), jnp.uint16).astype(jnp.uint32)
    g16 = lax.bitcast_convert_type(
        center[:, 1].astype(jnp.bfloat16), jnp.uint16).astype(jnp.uint32)
    crg = lax.bitcast_convert_type(r16 | (g16 << 16), jnp.int32)
    cbf = center[:, 2]
    output, label = _cluster_sc(input, index, crg, cbf)
    return (output, label)
